# baseline (speedup 1.0000x reference)
import functools

import jax
import jax.numpy as jnp
from jax import lax
from jax.experimental import pallas as pl
from jax.experimental.pallas import tpu as pltpu
from jax.experimental.pallas import tpu_sc as plsc

NC, NS, L = 2, 16, 16
NW = NC * NS
K = 512
RPW = K // NW
C = 256
NCH = C // L
PB = 7
NP = 14
NT = RPW * PB
GR = 112
LVL_S = (200, 100, 50, 25)
LVL_SCALE = (0.25, 0.125, 0.0625, 0.03125)
LVL_BASE = (0, 80000, 100000, 105000)

T0 = (224.0 * (0.5 - 1e-6)) ** 2
T1 = (224.0 * (1.0 - 1e-6)) ** 2
T2 = (224.0 * (2.0 - 1e-6)) ** 2


def _bcast(ref, row, lane_vec):
    return plsc.load_gather(ref, [jnp.full((L,), row, jnp.int32), lane_vec])


def _sc_roialign(table, boxes_t):
    mesh = plsc.VectorSubcoreMesh(
        core_axis_name="c", subcore_axis_name="s", num_cores=NC, num_subcores=NS
    )

    @functools.partial(
        pl.kernel,
        out_type=jax.ShapeDtypeStruct((K, PB * PB * C), jnp.float32),
        mesh=mesh,
        compiler_params=pltpu.CompilerParams(needs_layout_passes=False),
        scratch_types=[
            pltpu.VMEM((4, L), jnp.float32),
            pltpu.VMEM((8, L), jnp.float32),
            pltpu.VMEM((2, L), jnp.int32),
            pltpu.VMEM((2, L), jnp.int32),
            pltpu.VMEM((2, L), jnp.float32),
            pltpu.VMEM((2, L), jnp.int32),
            pltpu.VMEM((2, L), jnp.float32),
            pltpu.VMEM((16, L), jnp.float32),
            pltpu.VMEM((8, L), jnp.int32),
            pltpu.VMEM((GR,), jnp.int32),
            pltpu.VMEM((GR,), jnp.int32),
            pltpu.VMEM((GR, C // 2), jnp.int32),
            pltpu.VMEM((GR, C // 2), jnp.int32),
            pltpu.VMEM((PB * PB * C,), jnp.float32),
            pltpu.SemaphoreType.DMA,
            pltpu.SemaphoreType.DMA,
        ],
    )
    def body(table_hbm, boxes_hbm, out_hbm,
             box_v, pf, pi, ybuf, wybuf, xbuf, wxbuf, wbuf, cstage, idx_a, idx_b,
             rows_a, rows_b, acc_v, gsem0, gsem1):
        cid = lax.axis_index("c")
        sid = lax.axis_index("s")
        wid = sid * NC + cid
        r0 = wid * RPW

        pltpu.sync_copy(boxes_hbm.at[pl.ds(r0, L)], box_v.at[0])
        pltpu.sync_copy(boxes_hbm.at[pl.ds(K + r0, L)], box_v.at[1])
        pltpu.sync_copy(boxes_hbm.at[pl.ds(2 * K + r0, L)], box_v.at[2])
        pltpu.sync_copy(boxes_hbm.at[pl.ds(3 * K + r0, L)], box_v.at[3])
        x0 = box_v[0]
        y0 = box_v[1]
        x1 = box_v[2]
        y1 = box_v[3]

        lanes = lax.iota(jnp.int32, L)
        lanesf = lanes.astype(jnp.float32)

        area = (x1 - x0) * (y1 - y0)
        one = jnp.full((L,), 1, jnp.int32)
        zero = jnp.full((L,), 0, jnp.int32)
        lvl = (
            jnp.where(area >= T0, one, zero)
            + jnp.where(area >= T1, one, zero)
            + jnp.where(area >= T2, one, zero)
        )
        scale = jnp.where(
            lvl == 0, LVL_SCALE[0],
            jnp.where(lvl == 1, LVL_SCALE[1],
                      jnp.where(lvl == 2, LVL_SCALE[2], LVL_SCALE[3])))
        sz_f = jnp.where(
            lvl == 0, float(LVL_S[0]),
            jnp.where(lvl == 1, float(LVL_S[1]),
                      jnp.where(lvl == 2, float(LVL_S[2]), float(LVL_S[3]))))
        sz_i = jnp.where(
            lvl == 0, LVL_S[0],
            jnp.where(lvl == 1, LVL_S[1],
                      jnp.where(lvl == 2, LVL_S[2], LVL_S[3])))
        base = jnp.where(
            lvl == 0, LVL_BASE[0],
            jnp.where(lvl == 1, LVL_BASE[1],
                      jnp.where(lvl == 2, LVL_BASE[2], LVL_BASE[3])))
        b_idx = jnp.where((lanes + r0) >= (K // 2), one, zero)
        base = base + b_idx * sz_i * sz_i

        rx0 = x0 * scale
        ry0 = y0 * scale
        roi_w = jnp.maximum(x1 * scale - rx0, 1.0)
        roi_h = jnp.maximum(y1 * scale - ry0, 1.0)
        pf[0] = rx0
        pf[1] = ry0
        pf[2] = roi_w / float(PB)
        pf[3] = roi_h / float(PB)
        pf[4] = sz_f
        pi[0] = base
        pi[1] = sz_i

        fsamp = lanesf * 0.5 + 0.25
        ptmask = lanes < NP

        def setup_roi(rt):
            fr = jnp.full((L,), rt, jnp.int32)
            brx0 = _bcast(pf, 0, fr)
            bry0 = _bcast(pf, 1, fr)
            bbw = _bcast(pf, 2, fr)
            bbh = _bcast(pf, 3, fr)
            bsf = _bcast(pf, 4, fr)
            bbase = plsc.load_gather(pi, [jnp.full((L,), 0, jnp.int32), fr])
            bsi = plsc.load_gather(pi, [jnp.full((L,), 1, jnp.int32), fr])

            xs = brx0 + fsamp * bbw
            xval = (xs >= -1.0) & (xs <= bsf) & ptmask
            xc = jnp.minimum(jnp.maximum(xs, 0.0), bsf - 1.0)
            xl = xc.astype(jnp.int32)
            lx = xc - xl.astype(jnp.float32)
            xbuf[0] = xl
            xbuf[1] = jnp.minimum(xl + 1, bsi - 1)
            xvf = jnp.where(xval, 1.0, 0.0)
            wxbuf[0] = (1.0 - lx) * xvf
            wxbuf[1] = lx * xvf

            ys = bry0 + fsamp * bbh
            yval = (ys >= -1.0) & (ys <= bsf) & ptmask
            yc = jnp.minimum(jnp.maximum(ys, 0.0), bsf - 1.0)
            yl = yc.astype(jnp.int32)
            ly = yc - yl.astype(jnp.float32)
            ybuf[0] = yl * bsi + bbase
            ybuf[1] = jnp.minimum(yl + 1, bsi - 1) * bsi + bbase
            yvf = jnp.where(yval, 0.25, 0.0)
            wybuf[0] = (1.0 - ly) * yvf
            wybuf[1] = ly * yvf

        def fire(t, slot, idx_v, rows_v, sem):
            rt = t // PB
            by = t - rt * PB

            @pl.when(by == 0)
            def _():
                setup_roi(rt)

            xlv = xbuf[0]
            xhv = xbuf[1]
            wxl = wxbuf[0]
            wxh = wxbuf[1]
            for p in range(2):
                fpy = jnp.full((L,), 2 * by + p, jnp.int32)
                bylw = _bcast(ybuf, 0, fpy)
                byhw = _bcast(ybuf, 1, fpy)
                cstage[p * 4 + 0] = bylw + xlv
                cstage[p * 4 + 1] = bylw + xhv
                cstage[p * 4 + 2] = byhw + xlv
                cstage[p * 4 + 3] = byhw + xhv
                bwyl = _bcast(wybuf, 0, fpy)
                bwyh = _bcast(wybuf, 1, fpy)
                wbuf[slot * 8 + p * 4 + 0] = bwyl * wxl
                wbuf[slot * 8 + p * 4 + 1] = bwyl * wxh
                wbuf[slot * 8 + p * 4 + 2] = bwyh * wxl
                wbuf[slot * 8 + p * 4 + 3] = bwyh * wxh
            giota = lax.iota(jnp.int32, L)
            for k in range(PB):
                g = giota + (16 * k)
                rowi = g // 14
                lanei = g - rowi * 14
                idx_v[pl.ds(k * L, L)] = plsc.load_gather(cstage, [rowi, lanei])
            pltpu.async_copy(table_hbm.at[idx_v], rows_v, sem)

        def drain(idx_v, rows_v, sem):
            pltpu.make_async_copy(table_hbm.at[idx_v], rows_v, sem).wait()

        himask = jnp.full((L,), -65536, jnp.int32)

        def compute(t, slot, rows_v):
            rt = t // PB
            by = t - rt * PB

            iota49 = lax.iota(jnp.int32, L) * (PB * PB)

            def bx_body(bx, carry):
                px0 = 2 * bx
                f0 = jnp.full((L,), px0, jnp.int32)
                f1 = f0 + 1
                w = []
                for p in range(2):
                    for corner in range(4):
                        row = slot * 8 + p * 4 + corner
                        w.append((_bcast(wbuf, row, f0), _bcast(wbuf, row, f1)))
                obin = by * PB + bx

                def ch_body(ch, carry2):
                    sl = pl.ds(ch * L, L)
                    slo = None
                    shi = None
                    for p in range(2):
                        for corner in range(4):
                            r_base = p * 56 + corner * 14 + px0
                            w0, w1 = w[p * 4 + corner]
                            a = rows_v[r_base, sl]
                            b = rows_v[r_base + 1, sl]
                            alo = plsc.bitcast(jnp.left_shift(a, 16), jnp.float32)
                            ahi = plsc.bitcast(a & himask, jnp.float32)
                            blo = plsc.bitcast(jnp.left_shift(b, 16), jnp.float32)
                            bhi = plsc.bitcast(b & himask, jnp.float32)
                            tlo = w0 * alo + w1 * blo
                            thi = w0 * ahi + w1 * bhi
                            slo = tlo if slo is None else slo + tlo
                            shi = thi if shi is None else shi + thi
                    off = obin * C + ch * 2 * L
                    acc_v[pl.ds(off, L)] = slo
                    acc_v[pl.ds(off + L, L)] = shi
                    return carry2

                lax.fori_loop(0, NCH // 2, ch_body, 0, unroll=4)
                return carry

            lax.fori_loop(0, PB, bx_body, 0)

            @pl.when(by == PB - 1)
            def _():
                pltpu.sync_copy(acc_v, out_hbm.at[r0 + rt])

        def u_body(u, carry):
            t0 = 4 * u
            fire(t0, 0, idx_a, rows_a, gsem0)
            fire(t0 + 1, 1, idx_b, rows_b, gsem1)
            drain(idx_a, rows_a, gsem0)
            compute(t0, 0, rows_a)
            fire(t0 + 2, 0, idx_a, rows_a, gsem0)
            drain(idx_b, rows_b, gsem1)
            compute(t0 + 1, 1, rows_b)
            fire(t0 + 3, 1, idx_b, rows_b, gsem1)
            drain(idx_a, rows_a, gsem0)
            compute(t0 + 2, 0, rows_a)
            drain(idx_b, rows_b, gsem1)
            compute(t0 + 3, 1, rows_b)
            return carry

        lax.fori_loop(0, NT // 4, u_body, 0)

    return body(table, boxes_t)


def kernel(feat0, feat1, feat2, feat3, proposals0, proposals1):
    def pack(f):
        u = lax.bitcast_convert_type(f, jnp.uint32)
        rnd = (u + jnp.uint32(0x7FFF) + ((u >> jnp.uint32(16)) & jnp.uint32(1))) >> jnp.uint32(16)
        w = rnd[:, : C // 2] | (rnd[:, C // 2:] << jnp.uint32(16))
        return lax.bitcast_convert_type(w, jnp.int32)

    table = jnp.concatenate(
        [jnp.transpose(pack(f), (0, 2, 3, 1)).reshape(-1, C // 2)
         for f in (feat0, feat1, feat2, feat3)],
        axis=0,
    )
    boxes_t = jnp.concatenate([proposals0, proposals1], axis=0).T.reshape(-1)
    out = _sc_roialign(table, boxes_t)
    out = out.reshape(K, PB * PB, C // 32, 2, L)
    out = out.transpose(0, 3, 2, 4, 1).reshape(K, C, PB, PB)
    return out


# speedup vs baseline: 1.0255x; 1.0255x over previous
import functools

import jax
import jax.numpy as jnp
from jax import lax
from jax.experimental import pallas as pl
from jax.experimental.pallas import tpu as pltpu
from jax.experimental.pallas import tpu_sc as plsc

NC, NS, L = 2, 16, 16
NW = NC * NS
K = 512
RPW = K // NW
C = 256
NCH = C // L
PB = 7
NP = 14
NT = RPW * PB
GR = 112
LVL_S = (200, 100, 50, 25)
LVL_SCALE = (0.25, 0.125, 0.0625, 0.03125)
LVL_BASE = (0, 80000, 100000, 105000)

T0 = (224.0 * (0.5 - 1e-6)) ** 2
T1 = (224.0 * (1.0 - 1e-6)) ** 2
T2 = (224.0 * (2.0 - 1e-6)) ** 2


def _bcast(ref, row, lane_vec):
    return plsc.load_gather(ref, [jnp.full((L,), row, jnp.int32), lane_vec])


def _sc_roialign(table, boxes_t):
    mesh = plsc.VectorSubcoreMesh(
        core_axis_name="c", subcore_axis_name="s", num_cores=NC, num_subcores=NS
    )

    @functools.partial(
        pl.kernel,
        out_type=jax.ShapeDtypeStruct((K, PB * PB * C), jnp.float32),
        mesh=mesh,
        compiler_params=pltpu.CompilerParams(needs_layout_passes=False),
        scratch_types=[
            pltpu.VMEM((4, L), jnp.float32),
            pltpu.VMEM((8, L), jnp.float32),
            pltpu.VMEM((2, L), jnp.int32),
            pltpu.VMEM((2, L), jnp.int32),
            pltpu.VMEM((2, L), jnp.float32),
            pltpu.VMEM((2, L), jnp.int32),
            pltpu.VMEM((2, L), jnp.float32),
            pltpu.VMEM((16, L), jnp.float32),
            pltpu.VMEM((8, L), jnp.int32),
            pltpu.VMEM((GR,), jnp.int32),
            pltpu.VMEM((GR,), jnp.int32),
            pltpu.VMEM((GR, C // 2), jnp.int32),
            pltpu.VMEM((GR, C // 2), jnp.int32),
            pltpu.VMEM((PB * PB * C,), jnp.float32),
            pltpu.SemaphoreType.DMA,
            pltpu.SemaphoreType.DMA,
        ],
    )
    def body(table_hbm, boxes_hbm, out_hbm,
             box_v, pf, pi, ybuf, wybuf, xbuf, wxbuf, wbuf, cstage, idx_a, idx_b,
             rows_a, rows_b, acc_v, gsem0, gsem1):
        cid = lax.axis_index("c")
        sid = lax.axis_index("s")
        wid = sid * NC + cid
        r0 = wid * RPW

        pltpu.sync_copy(boxes_hbm.at[pl.ds(r0, L)], box_v.at[0])
        pltpu.sync_copy(boxes_hbm.at[pl.ds(K + r0, L)], box_v.at[1])
        pltpu.sync_copy(boxes_hbm.at[pl.ds(2 * K + r0, L)], box_v.at[2])
        pltpu.sync_copy(boxes_hbm.at[pl.ds(3 * K + r0, L)], box_v.at[3])
        x0 = box_v[0]
        y0 = box_v[1]
        x1 = box_v[2]
        y1 = box_v[3]

        lanes = lax.iota(jnp.int32, L)
        lanesf = lanes.astype(jnp.float32)

        area = (x1 - x0) * (y1 - y0)
        one = jnp.full((L,), 1, jnp.int32)
        zero = jnp.full((L,), 0, jnp.int32)
        lvl = (
            jnp.where(area >= T0, one, zero)
            + jnp.where(area >= T1, one, zero)
            + jnp.where(area >= T2, one, zero)
        )
        scale = jnp.where(
            lvl == 0, LVL_SCALE[0],
            jnp.where(lvl == 1, LVL_SCALE[1],
                      jnp.where(lvl == 2, LVL_SCALE[2], LVL_SCALE[3])))
        sz_f = jnp.where(
            lvl == 0, float(LVL_S[0]),
            jnp.where(lvl == 1, float(LVL_S[1]),
                      jnp.where(lvl == 2, float(LVL_S[2]), float(LVL_S[3]))))
        sz_i = jnp.where(
            lvl == 0, LVL_S[0],
            jnp.where(lvl == 1, LVL_S[1],
                      jnp.where(lvl == 2, LVL_S[2], LVL_S[3])))
        base = jnp.where(
            lvl == 0, LVL_BASE[0],
            jnp.where(lvl == 1, LVL_BASE[1],
                      jnp.where(lvl == 2, LVL_BASE[2], LVL_BASE[3])))
        b_idx = jnp.where((lanes + r0) >= (K // 2), one, zero)
        base = base + b_idx * sz_i * sz_i

        rx0 = x0 * scale
        ry0 = y0 * scale
        roi_w = jnp.maximum(x1 * scale - rx0, 1.0)
        roi_h = jnp.maximum(y1 * scale - ry0, 1.0)
        pf[0] = rx0
        pf[1] = ry0
        pf[2] = roi_w / float(PB)
        pf[3] = roi_h / float(PB)
        pf[4] = sz_f
        pi[0] = base
        pi[1] = sz_i

        fsamp = lanesf * 0.5 + 0.25
        ptmask = lanes < NP

        def setup_roi(rt):
            fr = jnp.full((L,), rt, jnp.int32)
            brx0 = _bcast(pf, 0, fr)
            bry0 = _bcast(pf, 1, fr)
            bbw = _bcast(pf, 2, fr)
            bbh = _bcast(pf, 3, fr)
            bsf = _bcast(pf, 4, fr)
            bbase = plsc.load_gather(pi, [jnp.full((L,), 0, jnp.int32), fr])
            bsi = plsc.load_gather(pi, [jnp.full((L,), 1, jnp.int32), fr])

            xs = brx0 + fsamp * bbw
            xval = (xs >= -1.0) & (xs <= bsf) & ptmask
            xc = jnp.minimum(jnp.maximum(xs, 0.0), bsf - 1.0)
            xl = xc.astype(jnp.int32)
            lx = xc - xl.astype(jnp.float32)
            xbuf[0] = xl
            xbuf[1] = jnp.minimum(xl + 1, bsi - 1)
            xvf = jnp.where(xval, 1.0, 0.0)
            wxbuf[0] = (1.0 - lx) * xvf
            wxbuf[1] = lx * xvf

            ys = bry0 + fsamp * bbh
            yval = (ys >= -1.0) & (ys <= bsf) & ptmask
            yc = jnp.minimum(jnp.maximum(ys, 0.0), bsf - 1.0)
            yl = yc.astype(jnp.int32)
            ly = yc - yl.astype(jnp.float32)
            ybuf[0] = yl * bsi + bbase
            ybuf[1] = jnp.minimum(yl + 1, bsi - 1) * bsi + bbase
            yvf = jnp.where(yval, 0.25, 0.0)
            wybuf[0] = (1.0 - ly) * yvf
            wybuf[1] = ly * yvf

        def fire(t, slot, idx_v, rows_v, sem):
            rt = t // PB
            by = t - rt * PB

            @pl.when(by == 0)
            def _():
                setup_roi(rt)

            xlv = xbuf[0]
            xhv = xbuf[1]
            wxl = wxbuf[0]
            wxh = wxbuf[1]
            for p in range(2):
                fpy = jnp.full((L,), 2 * by + p, jnp.int32)
                bylw = _bcast(ybuf, 0, fpy)
                byhw = _bcast(ybuf, 1, fpy)
                cstage[p * 4 + 0] = bylw + xlv
                cstage[p * 4 + 1] = bylw + xhv
                cstage[p * 4 + 2] = byhw + xlv
                cstage[p * 4 + 3] = byhw + xhv
                bwyl = _bcast(wybuf, 0, fpy)
                bwyh = _bcast(wybuf, 1, fpy)
                wbuf[slot * 8 + p * 4 + 0] = bwyl * wxl
                wbuf[slot * 8 + p * 4 + 1] = bwyl * wxh
                wbuf[slot * 8 + p * 4 + 2] = bwyh * wxl
                wbuf[slot * 8 + p * 4 + 3] = bwyh * wxh
            giota = lax.iota(jnp.int32, L)
            for k in range(PB):
                g = giota + (16 * k)
                rowi = g // 14
                lanei = g - rowi * 14
                idx_v[pl.ds(k * L, L)] = plsc.load_gather(cstage, [rowi, lanei])
            pltpu.async_copy(table_hbm.at[idx_v], rows_v, sem)

        def drain(idx_v, rows_v, sem):
            pltpu.make_async_copy(table_hbm.at[idx_v], rows_v, sem).wait()

        himask = jnp.full((L,), -65536, jnp.int32)

        def compute(t, slot, rows_v):
            rt = t // PB
            by = t - rt * PB

            iota49 = lax.iota(jnp.int32, L) * (PB * PB)

            def bx_body(bx, carry):
                px0 = 2 * bx
                f0 = jnp.full((L,), px0, jnp.int32)
                f1 = f0 + 1
                w = []
                for p in range(2):
                    for corner in range(4):
                        row = slot * 8 + p * 4 + corner
                        w.append((_bcast(wbuf, row, f0), _bcast(wbuf, row, f1)))
                obin = by * PB + bx

                def ch_body(ch, carry2):
                    sl = pl.ds(ch * L, L)
                    slo = None
                    shi = None
                    for p in range(2):
                        for corner in range(4):
                            r_base = p * 56 + corner * 14 + px0
                            w0, w1 = w[p * 4 + corner]
                            a = rows_v[r_base, sl]
                            b = rows_v[r_base + 1, sl]
                            alo = plsc.bitcast(jnp.left_shift(a, 16), jnp.float32)
                            ahi = plsc.bitcast(a & himask, jnp.float32)
                            blo = plsc.bitcast(jnp.left_shift(b, 16), jnp.float32)
                            bhi = plsc.bitcast(b & himask, jnp.float32)
                            tlo = w0 * alo + w1 * blo
                            thi = w0 * ahi + w1 * bhi
                            slo = tlo if slo is None else slo + tlo
                            shi = thi if shi is None else shi + thi
                    off = obin * C + ch * 2 * L
                    acc_v[pl.ds(off, L)] = slo
                    acc_v[pl.ds(off + L, L)] = shi
                    return carry2

                lax.fori_loop(0, NCH // 2, ch_body, 0, unroll=2)
                return carry

            lax.fori_loop(0, PB, bx_body, 0)

            @pl.when(by == PB - 1)
            def _():
                pltpu.sync_copy(acc_v, out_hbm.at[r0 + rt])

        def u_body(u, carry):
            t0 = 4 * u
            fire(t0, 0, idx_a, rows_a, gsem0)
            fire(t0 + 1, 1, idx_b, rows_b, gsem1)
            drain(idx_a, rows_a, gsem0)
            compute(t0, 0, rows_a)
            fire(t0 + 2, 0, idx_a, rows_a, gsem0)
            drain(idx_b, rows_b, gsem1)
            compute(t0 + 1, 1, rows_b)
            fire(t0 + 3, 1, idx_b, rows_b, gsem1)
            drain(idx_a, rows_a, gsem0)
            compute(t0 + 2, 0, rows_a)
            drain(idx_b, rows_b, gsem1)
            compute(t0 + 3, 1, rows_b)
            return carry

        lax.fori_loop(0, NT // 4, u_body, 0)

    return body(table, boxes_t)


def kernel(feat0, feat1, feat2, feat3, proposals0, proposals1):
    def pack(f):
        u = lax.bitcast_convert_type(f, jnp.uint32)
        rnd = (u + jnp.uint32(0x7FFF) + ((u >> jnp.uint32(16)) & jnp.uint32(1))) >> jnp.uint32(16)
        w = rnd[:, : C // 2] | (rnd[:, C // 2:] << jnp.uint32(16))
        return lax.bitcast_convert_type(w, jnp.int32)

    table = jnp.concatenate(
        [jnp.transpose(pack(f), (0, 2, 3, 1)).reshape(-1, C // 2)
         for f in (feat0, feat1, feat2, feat3)],
        axis=0,
    )
    boxes_t = jnp.concatenate([proposals0, proposals1], axis=0).T.reshape(-1)
    out = _sc_roialign(table, boxes_t)
    out = out.reshape(K, PB * PB, C // 32, 2, L)
    out = out.transpose(0, 3, 2, 4, 1).reshape(K, C, PB, PB)
    return out
